# baseline (speedup 1.0000x reference)
import functools

import jax
import jax.numpy as jnp
from jax import lax
from jax.experimental import pallas as pl
from jax.experimental.pallas import tpu as pltpu
from jax.experimental.pallas import tpu_sc as plsc

B = 4096
H = 50
D = 64
PD = 128
NC, NS = 2, 16
NW = NC * NS
TPW = B // NW
TPC = 4
RPC = TPC * H
NCHUNK = TPW // TPC
NBUF = 4
LOOK = 2

_mesh = plsc.VectorSubcoreMesh(core_axis_name="c", subcore_axis_name="s")


@functools.partial(
    pl.kernel,
    mesh=_mesh,
    out_type=jax.ShapeDtypeStruct((B * H, PD), jnp.float32),
    scratch_types=[
        pltpu.VMEM((TPW, PD), jnp.int32),
        pltpu.VMEM((NBUF, RPC, PD), jnp.float32),
        [pltpu.SemaphoreType.DMA] * NBUF,
        [pltpu.SemaphoreType.DMA] * NBUF,
    ],
    compiler_params=pltpu.CompilerParams(use_tc_tiling_on_sc=True),
)
def _emb_lookup(idx_hbm, table_hbm, out_hbm, idx_v, rows_v, gsems, wsems):
    wid = lax.axis_index("s") * NC + lax.axis_index("c")
    tok0 = wid * TPW
    row0 = tok0 * H

    pltpu.sync_copy(idx_hbm.at[pl.ds(tok0, TPW)], idx_v)

    def fire_gathers(c, b):
        for j in range(TPC):
            pltpu.async_copy(
                table_hbm.at[idx_v.at[c * TPC + j, pl.ds(0, H)]],
                rows_v.at[b, pl.ds(j * H, H)], gsems[b])

    def wait_gathers(c, b):
        for j in range(TPC):
            pltpu.make_async_copy(
                table_hbm.at[idx_v.at[c * TPC + j, pl.ds(0, H)]],
                rows_v.at[b, pl.ds(j * H, H)], gsems[b]).wait()

    def fire_writeback(c, b):
        pltpu.async_copy(rows_v.at[b],
                         out_hbm.at[pl.ds(row0 + c * RPC, RPC)], wsems[b])

    def wait_writeback(c, b):
        pltpu.make_async_copy(rows_v.at[b],
                              out_hbm.at[pl.ds(row0 + c * RPC, RPC)],
                              wsems[b]).wait()

    for b in range(LOOK):
        fire_gathers(b, b)

    def step(c, b):
        wait_gathers(c, b)
        fire_writeback(c, b)
        n = c + LOOK
        bn = (b + LOOK) % NBUF

        @pl.when(n < NCHUNK)
        def _():
            @pl.when(n >= NBUF)
            def _():
                wait_writeback(n - NBUF, bn)

            fire_gathers(n, bn)
        return 0

    lax.fori_loop(
        0, NCHUNK // NBUF,
        lambda i, x: [step(i * NBUF + b, b) for b in range(NBUF)][-1],
        0, unroll=False)

    for m in range(NCHUNK - NBUF, NCHUNK):
        wait_writeback(m, m % NBUF)


def kernel(token_ids, weight):
    wpad = jnp.pad(weight, ((0, 0), (0, PD - D)))
    idx = jnp.pad(token_ids.astype(jnp.int32), ((0, 0), (0, PD - H)))
    out = _emb_lookup(idx, wpad)
    return out.reshape(B, H, PD)[:, :, :D]


# speedup vs baseline: 1.5336x; 1.5336x over previous
import functools

import jax
import jax.numpy as jnp
from jax import lax
from jax.experimental import pallas as pl
from jax.experimental.pallas import tpu as pltpu
from jax.experimental.pallas import tpu_sc as plsc

B = 4096
H = 50
D = 64
PD = 128
NC, NS = 2, 16
NW = NC * NS
TPW = B // NW
TPC = 2
NCHUNK = TPW // TPC
NBUF = 4
LOOK = 2

_mesh = plsc.VectorSubcoreMesh(core_axis_name="c", subcore_axis_name="s")


@functools.partial(
    pl.kernel,
    mesh=_mesh,
    out_type=jax.ShapeDtypeStruct((B, H, PD), jnp.float32),
    scratch_types=[
        pltpu.VMEM((TPW, H), jnp.int32),
        pltpu.VMEM((NBUF, TPC, H, PD), jnp.float32),
        [pltpu.SemaphoreType.DMA] * NBUF,
        [pltpu.SemaphoreType.DMA] * NBUF,
    ],
    compiler_params=pltpu.CompilerParams(use_tc_tiling_on_sc=True),
)
def _emb_lookup(idx_hbm, table_hbm, out_hbm, idx_v, rows_v, gsems, wsems):
    wid = lax.axis_index("s") * NC + lax.axis_index("c")
    tok0 = wid * TPW

    pltpu.sync_copy(idx_hbm.at[pl.ds(tok0, TPW)], idx_v)

    def fire_gathers(c, b):
        for j in range(TPC):
            pltpu.async_copy(table_hbm.at[idx_v.at[c * TPC + j]],
                             rows_v.at[b, j], gsems[b])

    def wait_gathers(c, b):
        for j in range(TPC):
            pltpu.make_async_copy(table_hbm.at[idx_v.at[c * TPC + j]],
                                  rows_v.at[b, j], gsems[b]).wait()

    def fire_writeback(c, b):
        pltpu.async_copy(rows_v.at[b],
                         out_hbm.at[pl.ds(tok0 + c * TPC, TPC)], wsems[b])

    def wait_writeback(c, b):
        pltpu.make_async_copy(rows_v.at[b],
                              out_hbm.at[pl.ds(tok0 + c * TPC, TPC)],
                              wsems[b]).wait()

    for b in range(LOOK):
        fire_gathers(b, b)

    def step(c, b):
        wait_gathers(c, b)
        fire_writeback(c, b)
        n = c + LOOK
        bn = (b + LOOK) % NBUF

        @pl.when(n < NCHUNK)
        def _():
            @pl.when(n >= NBUF)
            def _():
                wait_writeback(n - NBUF, bn)

            fire_gathers(n, bn)
        return 0

    lax.fori_loop(
        0, NCHUNK // NBUF,
        lambda i, x: [step(i * NBUF + b, b) for b in range(NBUF)][-1],
        0, unroll=False)

    for m in range(NCHUNK - NBUF, NCHUNK):
        wait_writeback(m, m % NBUF)


def kernel(token_ids, weight):
    wpad = jnp.pad(weight, ((0, 0), (0, PD - D)))
    out = _emb_lookup(token_ids.astype(jnp.int32), wpad)
    return out[:, :, :D]


# speedup vs baseline: 1.5337x; 1.0001x over previous
import functools

import jax
import jax.numpy as jnp
from jax import lax
from jax.experimental import pallas as pl
from jax.experimental.pallas import tpu as pltpu
from jax.experimental.pallas import tpu_sc as plsc

B = 4096
H = 50
D = 64
PD = 128
NC, NS = 2, 16
NW = NC * NS
TPW = B // NW
TPC = 2
NCHUNK = TPW // TPC
NBUF = 4
LOOK = 3

_mesh = plsc.VectorSubcoreMesh(core_axis_name="c", subcore_axis_name="s")


@functools.partial(
    pl.kernel,
    mesh=_mesh,
    out_type=jax.ShapeDtypeStruct((B, H, PD), jnp.float32),
    scratch_types=[
        pltpu.VMEM((TPW, H), jnp.int32),
        pltpu.VMEM((NBUF, TPC, H, PD), jnp.float32),
        [pltpu.SemaphoreType.DMA] * NBUF,
        [pltpu.SemaphoreType.DMA] * NBUF,
    ],
    compiler_params=pltpu.CompilerParams(use_tc_tiling_on_sc=True),
)
def _emb_lookup(idx_hbm, table_hbm, out_hbm, idx_v, rows_v, gsems, wsems):
    wid = lax.axis_index("s") * NC + lax.axis_index("c")
    tok0 = wid * TPW

    pltpu.sync_copy(idx_hbm.at[pl.ds(tok0, TPW)], idx_v)

    def fire_gathers(c, b):
        for j in range(TPC):
            pltpu.async_copy(table_hbm.at[idx_v.at[c * TPC + j]],
                             rows_v.at[b, j], gsems[b])

    def wait_gathers(c, b):
        for j in range(TPC):
            pltpu.make_async_copy(table_hbm.at[idx_v.at[c * TPC + j]],
                                  rows_v.at[b, j], gsems[b]).wait()

    def fire_writeback(c, b):
        pltpu.async_copy(rows_v.at[b],
                         out_hbm.at[pl.ds(tok0 + c * TPC, TPC)], wsems[b])

    def wait_writeback(c, b):
        pltpu.make_async_copy(rows_v.at[b],
                              out_hbm.at[pl.ds(tok0 + c * TPC, TPC)],
                              wsems[b]).wait()

    for b in range(LOOK):
        fire_gathers(b, b)

    def step(c, b):
        wait_gathers(c, b)
        fire_writeback(c, b)
        n = c + LOOK
        bn = (b + LOOK) % NBUF

        @pl.when(n < NCHUNK)
        def _():
            @pl.when(n >= NBUF)
            def _():
                wait_writeback(n - NBUF, bn)

            fire_gathers(n, bn)
        return 0

    lax.fori_loop(
        0, NCHUNK // NBUF,
        lambda i, x: [step(i * NBUF + b, b) for b in range(NBUF)][-1],
        0, unroll=False)

    for m in range(NCHUNK - NBUF, NCHUNK):
        wait_writeback(m, m % NBUF)


def kernel(token_ids, weight):
    wpad = jnp.pad(weight, ((0, 0), (0, PD - D)))
    out = _emb_lookup(token_ids.astype(jnp.int32), wpad)
    return out[:, :, :D]


# speedup vs baseline: 1.5344x; 1.0004x over previous
import functools

import jax
import jax.numpy as jnp
from jax import lax
from jax.experimental import pallas as pl
from jax.experimental.pallas import tpu as pltpu
from jax.experimental.pallas import tpu_sc as plsc

B = 4096
H = 50
D = 64
PD = 128
NC, NS = 2, 16
NW = NC * NS
TPW = B // NW
TPC = 2
NCHUNK = TPW // TPC
NBUF = 4
LOOK = 2

_mesh = plsc.VectorSubcoreMesh(core_axis_name="c", subcore_axis_name="s")


@functools.partial(
    pl.kernel,
    mesh=_mesh,
    out_type=jax.ShapeDtypeStruct((B, H, PD), jnp.float32),
    scratch_types=[
        pltpu.VMEM((TPW, H), jnp.int32),
        pltpu.VMEM((NBUF, TPC, H, PD), jnp.float32),
        [pltpu.SemaphoreType.DMA] * NBUF,
        [pltpu.SemaphoreType.DMA] * NBUF,
    ],
    compiler_params=pltpu.CompilerParams(use_tc_tiling_on_sc=True),
)
def _emb_lookup(idx_hbm, table_hbm, out_hbm, idx_v, rows_v, gsems, wsems):
    wid = lax.axis_index("s") * NC + lax.axis_index("c")
    tok0 = wid * TPW

    pltpu.sync_copy(idx_hbm.at[pl.ds(tok0, TPW)], idx_v)

    def fire_gathers(c, b):
        for j in range(TPC):
            pltpu.async_copy(table_hbm.at[idx_v.at[c * TPC + j]],
                             rows_v.at[b, j], gsems[b])

    def wait_gathers(c, b):
        for j in range(TPC):
            pltpu.make_async_copy(table_hbm.at[idx_v.at[c * TPC + j]],
                                  rows_v.at[b, j], gsems[b]).wait()

    def fire_writeback(c, b):
        pltpu.async_copy(rows_v.at[b],
                         out_hbm.at[pl.ds(tok0 + c * TPC, TPC)], wsems[b])

    def wait_writeback(c, b):
        pltpu.make_async_copy(rows_v.at[b],
                              out_hbm.at[pl.ds(tok0 + c * TPC, TPC)],
                              wsems[b]).wait()

    for b in range(LOOK):
        fire_gathers(b, b)

    def step(c, b):
        wait_gathers(c, b)
        fire_writeback(c, b)
        n = c + LOOK
        bn = (b + LOOK) % NBUF

        @pl.when(n < NCHUNK)
        def _():
            @pl.when(n >= NBUF)
            def _():
                wait_writeback(n - NBUF, bn)

            fire_gathers(n, bn)
        return 0

    lax.fori_loop(
        0, NCHUNK // NBUF,
        lambda i, x: [step(i * NBUF + b, b) for b in range(NBUF)][-1],
        0, unroll=False)

    for m in range(NCHUNK - NBUF, NCHUNK):
        wait_writeback(m, m % NBUF)


def kernel(token_ids, weight):
    wpad = jnp.pad(weight, ((0, 0), (0, PD - D)))
    out = _emb_lookup(token_ids.astype(jnp.int32), wpad)
    return out[:, :, :D]
